# baseline (speedup 1.0000x reference)
import functools

import jax
import jax.numpy as jnp
from jax import lax
from jax.experimental import pallas as pl

_K = 16
_INF = float("inf")
_BIGI = 2**30


def _block_body(posT_ref, pospad_ref, q_ref, xfull_ref, xblk_ref,
                w1_ref, w2_ref, wef_ref, prm_ref, out_ref, *, bsz, n, k):
    i = pl.program_id(0)
    q = q_ref[...]
    C = posT_ref.shape[1]

    dot = functools.partial(lax.dot_general,
                            preferred_element_type=jnp.float32)
    mm = lambda a, bb: dot(a, bb, (((1,), (0,)), ((), ())))

    pT = posT_ref[...]
    qsq = (q[:, 0:1] * q[:, 0:1] + q[:, 1:2] * q[:, 1:2]) + q[:, 2:3] * q[:, 2:3]
    psq = (pT[0:1, :] * pT[0:1, :] + pT[1:2, :] * pT[1:2, :]) + pT[2:3, :] * pT[2:3, :]
    qp = lax.dot_general(q.astype(jnp.bfloat16), pT.astype(jnp.bfloat16),
                         (((1,), (0,)), ((), ())),
                         preferred_element_type=jnp.float32)
    d2 = qsq - 2.0 * qp + psq
    rows_g = i * bsz + lax.broadcasted_iota(jnp.int32, (bsz, C), 0)
    cols = lax.broadcasted_iota(jnp.int32, (bsz, C), 1)
    d2 = jnp.where(cols == rows_g, _INF, d2)

    d2dir = ((q[:, 0:1] - pT[0:1, :]) ** 2
             + (q[:, 1:2] - pT[1:2, :]) ** 2
             + (q[:, 2:3] - pT[2:3, :]) ** 2)

    for _ in range(k):
        c = jnp.argmin(d2, axis=1).astype(jnp.int32).reshape(bsz, 1)
        d2 = jnp.where(cols == c, _INF, d2)
    mask = jnp.where((d2 == _INF) & (cols != rows_g), 1.0, 0.0)
    dsum = jnp.sum(mask * jnp.sqrt(d2dir), axis=1, keepdims=True)

    inv_k = jnp.float32(1.0 / k)
    x_nbr = mm(mask, xfull_ref[...]) * inv_k
    rel8 = mm(mask, pospad_ref[...]) * inv_k - q

    h = (mm(xblk_ref[...], w1_ref[...])
         + mm(x_nbr, w2_ref[...])
         + mm(rel8, wef_ref[...])
         + (dsum * inv_k) * prm_ref[3:4, :]
         + prm_ref[0:1, :])

    mu = jnp.mean(h, axis=1, keepdims=True)
    hc = h - mu
    var = jnp.mean(hc * hc, axis=1, keepdims=True)
    h = hc / jnp.sqrt(var + 1e-5) * prm_ref[1:2, :] + prm_ref[2:3, :]
    out_ref[...] = jnp.maximum(h, 0.0)


def kernel(x, pos, W, b, gamma, beta):
    n, D = x.shape
    k = min(_K, n - 1)
    C = ((n + 127) // 128) * 128
    bsz = next(bb for bb in (200, 80, 40, 16, 8, 4, 2, 1) if n % bb == 0)

    pospad = jnp.zeros((C, 8), jnp.float32)
    pospad = pospad.at[:n, :3].set(pos).at[n:, :3].set(1e4)
    posT8 = pospad.T
    xpad = jnp.zeros((C, D), jnp.float32).at[:n].set(x)
    W1T = W[:, :D].T
    W2T = W[:, D:2 * D].T
    Wef = jnp.zeros((8, D), jnp.float32).at[:3].set(W[:, 2 * D:2 * D + 3].T)
    prm = jnp.zeros((8, D), jnp.float32)
    prm = prm.at[0].set(b).at[1].set(gamma).at[2].set(beta)
    prm = prm.at[3].set(W[:, 2 * D + 3])

    grid = (n // bsz,)
    full = lambda shp: pl.BlockSpec(shp, lambda i: (0, 0))
    blk = lambda shp: pl.BlockSpec(shp, lambda i: (i, 0))

    return pl.pallas_call(
        functools.partial(_block_body, bsz=bsz, n=n, k=k),
        grid=grid,
        in_specs=[
            full((8, C)),
            full((C, 8)),
            blk((bsz, 8)),
            full((C, D)),
            blk((bsz, D)),
            full((D, D)),
            full((D, D)),
            full((8, D)),
            full((8, D)),
        ],
        out_specs=blk((bsz, D)),
        out_shape=jax.ShapeDtypeStruct((n, D), jnp.float32),
    )(posT8, pospad, pospad, xpad, x, W1T, W2T, Wef, prm)


# speedup vs baseline: 3.2314x; 3.2314x over previous
import functools

import jax
import jax.numpy as jnp
from jax import lax
from jax.experimental import pallas as pl

_K = 16
_INF = float("inf")
_BIGI = 2**30


def _block_body(posT_ref, pospad_ref, q_ref, xfull_ref, xblk_ref,
                w1_ref, w2_ref, wef_ref, prm_ref, out_ref, *, bsz, n, k):
    i = pl.program_id(0)
    q = q_ref[...]
    C = posT_ref.shape[1]

    dot = functools.partial(lax.dot_general,
                            preferred_element_type=jnp.float32)
    mm = lambda a, bb: dot(a, bb, (((1,), (0,)), ((), ())))

    pT = posT_ref[...]
    qsq = (q[:, 0:1] * q[:, 0:1] + q[:, 1:2] * q[:, 1:2]) + q[:, 2:3] * q[:, 2:3]
    psq = (pT[0:1, :] * pT[0:1, :] + pT[1:2, :] * pT[1:2, :]) + pT[2:3, :] * pT[2:3, :]
    qp = lax.dot_general(q.astype(jnp.bfloat16), pT.astype(jnp.bfloat16),
                         (((1,), (0,)), ((), ())),
                         preferred_element_type=jnp.float32)
    d2 = qsq - 2.0 * qp + psq
    rows_g = i * bsz + lax.broadcasted_iota(jnp.int32, (bsz, C), 0)
    cols = lax.broadcasted_iota(jnp.int32, (bsz, C), 1)
    d2 = jnp.where(cols == rows_g, _INF, d2)

    d2dir = ((q[:, 0:1] - pT[0:1, :]) ** 2
             + (q[:, 1:2] - pT[1:2, :]) ** 2
             + (q[:, 2:3] - pT[2:3, :]) ** 2)

    nslab = C // 128
    lvls = [jnp.full((bsz, 128), _INF, jnp.float32) for _ in range(4)]
    for s in range(nslab):
        xv = d2[:, s * 128:(s + 1) * 128]
        for j in range(4):
            lo = jnp.minimum(lvls[j], xv)
            xv = jnp.maximum(lvls[j], xv)
            lvls[j] = lo
    S = jnp.concatenate(lvls, axis=1)
    t16 = jnp.zeros((bsz, 1), jnp.float32)
    for _ in range(k):
        t16 = jnp.min(S, axis=1, keepdims=True)
        S = jnp.where(S == t16, _INF, S)
    mask = jnp.where(d2 <= t16, 1.0, 0.0)
    dsum = jnp.sum(mask * jnp.sqrt(d2dir), axis=1, keepdims=True)

    inv_k = jnp.float32(1.0 / k)
    x_nbr = mm(mask, xfull_ref[...]) * inv_k
    rel8 = mm(mask, pospad_ref[...]) * inv_k - q

    h = (mm(xblk_ref[...], w1_ref[...])
         + mm(x_nbr, w2_ref[...])
         + mm(rel8, wef_ref[...])
         + (dsum * inv_k) * prm_ref[3:4, :]
         + prm_ref[0:1, :])

    mu = jnp.mean(h, axis=1, keepdims=True)
    hc = h - mu
    var = jnp.mean(hc * hc, axis=1, keepdims=True)
    h = hc / jnp.sqrt(var + 1e-5) * prm_ref[1:2, :] + prm_ref[2:3, :]
    out_ref[...] = jnp.maximum(h, 0.0)


def kernel(x, pos, W, b, gamma, beta):
    n, D = x.shape
    k = min(_K, n - 1)
    C = ((n + 127) // 128) * 128
    bsz = next(bb for bb in (80, 40, 16, 8, 4, 2, 1) if n % bb == 0)

    pospad = jnp.zeros((C, 8), jnp.float32)
    pospad = pospad.at[:n, :3].set(pos).at[n:, :3].set(1e4)
    posT8 = pospad.T
    xpad = jnp.zeros((C, D), jnp.float32).at[:n].set(x)
    W1T = W[:, :D].T
    W2T = W[:, D:2 * D].T
    Wef = jnp.zeros((8, D), jnp.float32).at[:3].set(W[:, 2 * D:2 * D + 3].T)
    prm = jnp.zeros((8, D), jnp.float32)
    prm = prm.at[0].set(b).at[1].set(gamma).at[2].set(beta)
    prm = prm.at[3].set(W[:, 2 * D + 3])

    grid = (n // bsz,)
    full = lambda shp: pl.BlockSpec(shp, lambda i: (0, 0))
    blk = lambda shp: pl.BlockSpec(shp, lambda i: (i, 0))

    return pl.pallas_call(
        functools.partial(_block_body, bsz=bsz, n=n, k=k),
        grid=grid,
        in_specs=[
            full((8, C)),
            full((C, 8)),
            blk((bsz, 8)),
            full((C, D)),
            blk((bsz, D)),
            full((D, D)),
            full((D, D)),
            full((8, D)),
            full((8, D)),
        ],
        out_specs=blk((bsz, D)),
        out_shape=jax.ShapeDtypeStruct((n, D), jnp.float32),
    )(posT8, pospad, pospad, xpad, x, W1T, W2T, Wef, prm)
